# baseline (speedup 1.0000x reference)
import functools

import jax
import jax.numpy as jnp
from jax import lax
from jax.experimental import pallas as pl
from jax.experimental.pallas import tpu as pltpu

N = 8192
D = 768
E = 64
K = 8
BLK = 256
GRID = N // BLK


def _router_body(x_ref, w1_ref, b1_ref, w2_ref, b2_ref, keys_ref,
                 idx_ref, scr_ref, imp_ref, load_ref, loss_ref, kn_ref,
                 sel_ref):
    i = pl.program_id(0)

    @pl.when(i == 0)
    def _init():
        keys = keys_ref[...]
        kn_ref[...] = keys / jnp.maximum(
            jnp.sqrt(jnp.sum(keys * keys, axis=1, keepdims=True)), 1e-12)
        imp_ref[...] = jnp.zeros_like(imp_ref)
        load_ref[...] = jnp.zeros_like(load_ref)

    kn = kn_ref[...]

    x = x_ref[...]
    h = lax.dot_general(x, w1_ref[...], (((1,), (1,)), ((), ())),
                        preferred_element_type=jnp.float32)
    h = jnp.maximum(h + b1_ref[...], 0.0)
    q = lax.dot_general(h, w2_ref[...], (((1,), (1,)), ((), ())),
                        preferred_element_type=jnp.float32)
    q = q + b2_ref[...]

    ones_row = jnp.ones((1, D), jnp.float32)
    qsq_row = lax.dot_general(ones_row, q * q, (((1,), (1,)), ((), ())),
                              preferred_element_type=jnp.float32)
    inv_nq = 1.0 / jnp.maximum(jnp.sqrt(qsq_row), 1e-12)

    logits_t = lax.dot_general(kn, q, (((1,), (1,)), ((), ())),
                               preferred_element_type=jnp.float32) * inv_nq

    eidx_t = lax.broadcasted_iota(jnp.int32, (E, BLK), 0)
    work = logits_t
    vals = []
    idxs = []
    for _ in range(K):
        m = jnp.max(work, axis=0, keepdims=True)
        a = jnp.min(jnp.where(work >= m, eidx_t, E), axis=0,
                    keepdims=True)
        vals.append(m)
        idxs.append(a)
        work = jnp.where(eidx_t == a, -jnp.inf, work)

    tv_t = jnp.concatenate(vals, axis=0)
    ti_t = jnp.concatenate(idxs, axis=0)

    ex = jnp.exp(tv_t - vals[0])
    scores_t = ex / jnp.sum(ex, axis=0, keepdims=True)

    eye_k = (lax.broadcasted_iota(jnp.int32, (K, K), 0) ==
             lax.broadcasted_iota(jnp.int32, (K, K), 1)).astype(jnp.float32)
    scores = lax.dot_general(scores_t, eye_k, (((0,), (0,)), ((), ())),
                             preferred_element_type=jnp.float32)
    idx_f = lax.dot_general(ti_t.astype(jnp.float32), eye_k,
                            (((0,), (0,)), ((), ())),
                            preferred_element_type=jnp.float32)
    idx_ref[...] = idx_f.astype(jnp.int32)
    scr_ref[...] = scores

    scf = jnp.zeros((E, BLK), jnp.float32)
    for j in range(K):
        oh_j = (eidx_t == idxs[j])
        scf = scf + jnp.where(oh_j, scores_t[j:j + 1, :], 0.0)
        sel_j = lax.dot_general(oh_j.astype(jnp.float32), kn,
                                (((0,), (0,)), ((), ())),
                                preferred_element_type=jnp.float32)
        sel_ref[:, j, :] = sel_j

    imp_ref[...] += jnp.sum(scf, axis=1, keepdims=True)
    load_ref[...] += jnp.sum((scf > 0).astype(jnp.int32), axis=1,
                             keepdims=True)

    @pl.when(i == GRID - 1)
    def _loss():
        def cv2(v):
            mean = jnp.sum(v) / E
            var = jnp.sum((v - mean) ** 2) / (E - 1)
            return var / (mean * mean + 1e-10)
        impf = imp_ref[...]
        loadf = load_ref[...].astype(jnp.float32)
        loss_ref[0, 0] = 0.01 * (cv2(impf) + cv2(loadf))


@functools.partial(jax.jit)
def _router(x, W1, b1, W2, b2, keys):
    out = pl.pallas_call(
        _router_body,
        grid=(GRID,),
        in_specs=[
            pl.BlockSpec((BLK, D), lambda i: (i, 0)),
            pl.BlockSpec((D, D), lambda i: (0, 0)),
            pl.BlockSpec((1, D), lambda i: (0, 0)),
            pl.BlockSpec((D, D), lambda i: (0, 0)),
            pl.BlockSpec((1, D), lambda i: (0, 0)),
            pl.BlockSpec((E, D), lambda i: (0, 0)),
        ],
        out_specs=[
            pl.BlockSpec((BLK, K), lambda i: (i, 0)),
            pl.BlockSpec((BLK, K), lambda i: (i, 0)),
            pl.BlockSpec((E, 1), lambda i: (0, 0)),
            pl.BlockSpec((E, 1), lambda i: (0, 0)),
            pl.BlockSpec(memory_space=pltpu.SMEM),
            pl.BlockSpec((E, D), lambda i: (0, 0)),
            pl.BlockSpec((BLK, K, D), lambda i: (i, 0, 0)),
        ],
        out_shape=[
            jax.ShapeDtypeStruct((N, K), jnp.int32),
            jax.ShapeDtypeStruct((N, K), jnp.float32),
            jax.ShapeDtypeStruct((E, 1), jnp.float32),
            jax.ShapeDtypeStruct((E, 1), jnp.int32),
            jax.ShapeDtypeStruct((1, 1), jnp.float32),
            jax.ShapeDtypeStruct((E, D), jnp.float32),
            jax.ShapeDtypeStruct((N, K, D), jnp.float32),
        ],
    )(x, W1, b1.reshape(1, D), W2, b2.reshape(1, D), keys)
    return out


def kernel(x, W1, b1, W2, b2, keys):
    idx, scores, imp2, load2, loss2, kn, sel = _router(
        x, W1, b1, W2, b2, keys)
    return (idx, scores, loss2[0, 0], load2[:, 0], imp2[:, 0], sel)


# speedup vs baseline: 1.4668x; 1.4668x over previous
import functools

import jax
import jax.numpy as jnp
from jax import lax
from jax.experimental import pallas as pl
from jax.experimental.pallas import tpu as pltpu

N = 8192
D = 768
E = 64
K = 8
BLK = 256
GRID = N // BLK


def _router_body(x_ref, w1_ref, b1_ref, w2_ref, b2_ref, keys_ref,
                 idx_ref, scr_ref, imp_ref, load_ref, loss_ref, kn_ref,
                 sel_ref):
    i = pl.program_id(0)

    @pl.when(i == 0)
    def _init():
        imp_ref[...] = jnp.zeros_like(imp_ref)
        load_ref[...] = jnp.zeros_like(load_ref)

    ones_col = jnp.ones((D, 1), jnp.float32)
    keys = keys_ref[...]
    ksq_col = lax.dot_general(keys * keys, ones_col, (((1,), (0,)), ((), ())),
                              preferred_element_type=jnp.float32)
    kn = keys / jnp.maximum(jnp.sqrt(ksq_col), 1e-12)

    @pl.when(i == 0)
    def _kn_out():
        kn_ref[...] = kn

    x = x_ref[...]
    h = lax.dot_general(x, w1_ref[...], (((1,), (1,)), ((), ())),
                        preferred_element_type=jnp.float32)
    h = jnp.maximum(h + b1_ref[...], 0.0)
    q = lax.dot_general(h, w2_ref[...], (((1,), (1,)), ((), ())),
                        preferred_element_type=jnp.float32)
    q = q + b2_ref[...]

    ones_row = jnp.ones((1, D), jnp.float32)
    qsq_row = lax.dot_general(ones_row, q * q, (((1,), (1,)), ((), ())),
                              preferred_element_type=jnp.float32)
    inv_nq = 1.0 / jnp.maximum(jnp.sqrt(qsq_row), 1e-12)

    logits_t = lax.dot_general(kn, q, (((1,), (1,)), ((), ())),
                               preferred_element_type=jnp.float32) * inv_nq

    eidx_t = lax.broadcasted_iota(jnp.int32, (E, BLK), 0)
    work = logits_t
    vals = []
    idxs = []
    for _ in range(K):
        m = jnp.max(work, axis=0, keepdims=True)
        a = jnp.min(jnp.where(work >= m, eidx_t, E), axis=0,
                    keepdims=True)
        vals.append(m)
        idxs.append(a)
        work = jnp.where(eidx_t == a, -jnp.inf, work)

    tv_t = jnp.concatenate(vals, axis=0)
    ti_t = jnp.concatenate(idxs, axis=0)

    ex = jnp.exp(tv_t - vals[0])
    scores_t = ex / jnp.sum(ex, axis=0, keepdims=True)

    eye_k = (lax.broadcasted_iota(jnp.int32, (K, K), 0) ==
             lax.broadcasted_iota(jnp.int32, (K, K), 1)).astype(jnp.float32)
    scores = lax.dot_general(scores_t, eye_k, (((0,), (0,)), ((), ())),
                             preferred_element_type=jnp.float32)
    idx_f = lax.dot_general(ti_t.astype(jnp.float32), eye_k,
                            (((0,), (0,)), ((), ())),
                            preferred_element_type=jnp.float32)
    idx_ref[...] = idx_f.astype(jnp.int32)
    scr_ref[...] = scores

    scf = jnp.zeros((E, BLK), jnp.float32)
    for j in range(K):
        scf = scf + jnp.where(eidx_t == idxs[j], scores_t[j:j + 1, :], 0.0)

    top_idx = idx_f.astype(jnp.int32)
    oh = (top_idx[:, :, None] ==
          lax.broadcasted_iota(jnp.int32, (BLK, K, E), 2)).astype(jnp.float32)
    sel = lax.dot_general(oh.reshape(BLK * K, E), kn,
                          (((1,), (0,)), ((), ())),
                          preferred_element_type=jnp.float32)
    sel_ref[...] = sel.reshape(BLK, K, D)

    imp_ref[...] += jnp.sum(scf, axis=1, keepdims=True)
    load_ref[...] += jnp.sum((scf > 0).astype(jnp.int32), axis=1,
                             keepdims=True)

    @pl.when(i == GRID - 1)
    def _loss():
        def cv2(v):
            mean = jnp.sum(v) / E
            var = jnp.sum((v - mean) ** 2) / (E - 1)
            return var / (mean * mean + 1e-10)
        impf = imp_ref[...]
        loadf = load_ref[...].astype(jnp.float32)
        loss_ref[0, 0] = 0.01 * (cv2(impf) + cv2(loadf))


@functools.partial(jax.jit)
def _router(x, W1, b1, W2, b2, keys):
    out = pl.pallas_call(
        _router_body,
        grid=(GRID,),
        in_specs=[
            pl.BlockSpec((BLK, D), lambda i: (i, 0)),
            pl.BlockSpec((D, D), lambda i: (0, 0)),
            pl.BlockSpec((1, D), lambda i: (0, 0)),
            pl.BlockSpec((D, D), lambda i: (0, 0)),
            pl.BlockSpec((1, D), lambda i: (0, 0)),
            pl.BlockSpec((E, D), lambda i: (0, 0)),
        ],
        out_specs=[
            pl.BlockSpec((BLK, K), lambda i: (i, 0)),
            pl.BlockSpec((BLK, K), lambda i: (i, 0)),
            pl.BlockSpec((E, 1), lambda i: (0, 0)),
            pl.BlockSpec((E, 1), lambda i: (0, 0)),
            pl.BlockSpec(memory_space=pltpu.SMEM),
            pl.BlockSpec((E, D), lambda i: (0, 0)),
            pl.BlockSpec((BLK, K, D), lambda i: (i, 0, 0)),
        ],
        out_shape=[
            jax.ShapeDtypeStruct((N, K), jnp.int32),
            jax.ShapeDtypeStruct((N, K), jnp.float32),
            jax.ShapeDtypeStruct((E, 1), jnp.float32),
            jax.ShapeDtypeStruct((E, 1), jnp.int32),
            jax.ShapeDtypeStruct((1, 1), jnp.float32),
            jax.ShapeDtypeStruct((E, D), jnp.float32),
            jax.ShapeDtypeStruct((N, K, D), jnp.float32),
        ],
    )(x, W1, b1.reshape(1, D), W2, b2.reshape(1, D), keys)
    return out


def kernel(x, W1, b1, W2, b2, keys):
    idx, scores, imp2, load2, loss2, kn, sel = _router(
        x, W1, b1, W2, b2, keys)
    return (idx, scores, loss2[0, 0], load2[:, 0], imp2[:, 0], sel)


# speedup vs baseline: 1.5082x; 1.0283x over previous
import functools

import jax
import jax.numpy as jnp
from jax import lax
from jax.experimental import pallas as pl
from jax.experimental.pallas import tpu as pltpu

N = 8192
D = 768
E = 64
K = 8
BLK = 256
GRID = N // BLK


def _router_body(x_ref, w1_ref, b1_ref, w2_ref, b2_ref, keys_ref,
                 idx_ref, scr_ref, imp_ref, load_ref, loss_ref, kn_ref,
                 sel_ref):
    i = pl.program_id(0)

    @pl.when(i == 0)
    def _init():
        imp_ref[...] = jnp.zeros_like(imp_ref)
        load_ref[...] = jnp.zeros_like(load_ref)

    keys = keys_ref[...]
    kn = keys / jnp.maximum(
        jnp.sqrt(jnp.sum(keys * keys, axis=1, keepdims=True)), 1e-12)

    @pl.when(i == 0)
    def _kn_out():
        kn_ref[...] = kn

    x = x_ref[...]
    h = lax.dot_general(x, w1_ref[...], (((1,), (1,)), ((), ())),
                        preferred_element_type=jnp.float32)
    h = jnp.maximum(h + b1_ref[...], 0.0)
    q = lax.dot_general(h, w2_ref[...], (((1,), (1,)), ((), ())),
                        preferred_element_type=jnp.float32)
    q = q + b2_ref[...]

    ones_row = jnp.ones((1, D), jnp.float32)
    qsq_row = lax.dot_general(ones_row, q * q, (((1,), (1,)), ((), ())),
                              preferred_element_type=jnp.float32)
    inv_nq = 1.0 / jnp.maximum(jnp.sqrt(qsq_row), 1e-12)

    logits_t = lax.dot_general(kn, q, (((1,), (1,)), ((), ())),
                               preferred_element_type=jnp.float32) * inv_nq

    eidx_t = lax.broadcasted_iota(jnp.int32, (E, BLK), 0)
    work = logits_t
    vals = []
    idxs = []
    for _ in range(K):
        m = jnp.max(work, axis=0, keepdims=True)
        a = jnp.min(jnp.where(work >= m, eidx_t, E), axis=0,
                    keepdims=True)
        vals.append(m)
        idxs.append(a)
        work = jnp.where(eidx_t == a, -jnp.inf, work)

    tv_t = jnp.concatenate(vals, axis=0)
    ti_t = jnp.concatenate(idxs, axis=0)

    ex = jnp.exp(tv_t - vals[0])
    scores_t = ex / jnp.sum(ex, axis=0, keepdims=True)

    eye_k = (lax.broadcasted_iota(jnp.int32, (K, K), 0) ==
             lax.broadcasted_iota(jnp.int32, (K, K), 1)).astype(jnp.float32)
    scores = lax.dot_general(scores_t, eye_k, (((0,), (0,)), ((), ())),
                             preferred_element_type=jnp.float32)
    idx_f = lax.dot_general(ti_t.astype(jnp.float32), eye_k,
                            (((0,), (0,)), ((), ())),
                            preferred_element_type=jnp.float32)
    idx_ref[...] = idx_f.astype(jnp.int32)
    scr_ref[...] = scores

    scf = jnp.zeros((E, BLK), jnp.float32)
    for j in range(K):
        scf = scf + jnp.where(eidx_t == idxs[j], scores_t[j:j + 1, :], 0.0)

    top_idx = idx_f.astype(jnp.int32)
    oh = (top_idx[:, :, None] ==
          lax.broadcasted_iota(jnp.int32, (BLK, K, E), 2)).astype(jnp.float32)
    sel = lax.dot_general(oh.reshape(BLK * K, E), kn,
                          (((1,), (0,)), ((), ())),
                          preferred_element_type=jnp.float32)
    sel_ref[...] = sel.reshape(BLK, K, D)

    imp_ref[...] += jnp.sum(scf, axis=1, keepdims=True)
    load_ref[...] += jnp.sum((scf > 0).astype(jnp.int32), axis=1,
                             keepdims=True)

    @pl.when(i == GRID - 1)
    def _loss():
        def cv2(v):
            mean = jnp.sum(v) / E
            var = jnp.sum((v - mean) ** 2) / (E - 1)
            return var / (mean * mean + 1e-10)
        impf = imp_ref[...]
        loadf = load_ref[...].astype(jnp.float32)
        loss_ref[0, 0] = 0.01 * (cv2(impf) + cv2(loadf))


@functools.partial(jax.jit)
def _router(x, W1, b1, W2, b2, keys):
    out = pl.pallas_call(
        _router_body,
        grid=(GRID,),
        in_specs=[
            pl.BlockSpec((BLK, D), lambda i: (i, 0)),
            pl.BlockSpec((D, D), lambda i: (0, 0)),
            pl.BlockSpec((1, D), lambda i: (0, 0)),
            pl.BlockSpec((D, D), lambda i: (0, 0)),
            pl.BlockSpec((1, D), lambda i: (0, 0)),
            pl.BlockSpec((E, D), lambda i: (0, 0)),
        ],
        out_specs=[
            pl.BlockSpec((BLK, K), lambda i: (i, 0)),
            pl.BlockSpec((BLK, K), lambda i: (i, 0)),
            pl.BlockSpec((E, 1), lambda i: (0, 0)),
            pl.BlockSpec((E, 1), lambda i: (0, 0)),
            pl.BlockSpec(memory_space=pltpu.SMEM),
            pl.BlockSpec((E, D), lambda i: (0, 0)),
            pl.BlockSpec((BLK, K, D), lambda i: (i, 0, 0)),
        ],
        out_shape=[
            jax.ShapeDtypeStruct((N, K), jnp.int32),
            jax.ShapeDtypeStruct((N, K), jnp.float32),
            jax.ShapeDtypeStruct((E, 1), jnp.float32),
            jax.ShapeDtypeStruct((E, 1), jnp.int32),
            jax.ShapeDtypeStruct((1, 1), jnp.float32),
            jax.ShapeDtypeStruct((E, D), jnp.float32),
            jax.ShapeDtypeStruct((N, K, D), jnp.float32),
        ],
    )(x, W1, b1.reshape(1, D), W2, b2.reshape(1, D), keys)
    return out


def kernel(x, W1, b1, W2, b2, keys):
    idx, scores, imp2, load2, loss2, kn, sel = _router(
        x, W1, b1, W2, b2, keys)
    return (idx, scores, loss2[0, 0], load2[:, 0], imp2[:, 0], sel)
